# baseline (speedup 1.0000x reference)
import functools

import jax
import jax.numpy as jnp
from jax import lax
from jax.experimental import pallas as pl
from jax.experimental.pallas import tpu as pltpu
from jax.experimental.pallas import tpu_sc as plsc

_NUM_WORKERS = 32


def _cross_term(t, s):
    t16 = t.astype(jnp.bfloat16).astype(jnp.float32)
    s16 = s.astype(jnp.bfloat16).astype(jnp.float32)
    return ((t16[:, 0:1] * s16[0:1, :] + t16[:, 1:2] * s16[1:2, :])
            + t16[:, 2:3] * s16[2:3, :])


def _knn_body(tpos_ref, spos_ref, sbatch_ref, *out_refs, k, n_src):
    t = tpos_ref[...]
    s = spos_ref[...]
    sb = sbatch_ref[0:1, :]
    cross = _cross_term(t, s)
    ss_t = t[:, 0:1] * t[:, 0:1] + t[:, 1:2] * t[:, 1:2] + t[:, 2:3] * t[:, 2:3]
    ss_s = s[0:1, :] * s[0:1, :] + s[1:2, :] * s[1:2, :] + s[2:3, :] * s[2:3, :]
    d2 = (ss_t + ss_s) - 2.0 * cross
    d2 = jnp.maximum(d2, 0.0)
    d2 = jnp.where(t[:, 3:4] != sb, jnp.float32(jnp.inf), d2)
    lanes = lax.broadcasted_iota(jnp.int32, d2.shape, 1)
    dists = []
    for j in range(k):
        m = jnp.min(d2, axis=1, keepdims=True)
        eq = d2 == m
        idx = jnp.min(jnp.where(eq, lanes, n_src), axis=1, keepdims=True)
        out_refs[j][...] = idx
        dists.append(m)
        if j + 1 < k:
            d2 = jnp.where(lanes == idx, jnp.float32(jnp.inf), d2)
    if k > 1:
        ws = [1.0 / jnp.maximum(d, 1e-16) for d in dists]
        tot = ws[0]
        for w in ws[1:]:
            tot = tot + w
        for j in range(k):
            out_refs[k + j][...] = ws[j] / tot


def _knn(tpos, spos, sbatch, k, tb):
    n_tgt = tpos.shape[0]
    n_src = spos.shape[1]
    n_out = 2 * k if k > 1 else 1
    outs = ([jax.ShapeDtypeStruct((n_tgt, 1), jnp.int32)] * k
            + [jax.ShapeDtypeStruct((n_tgt, 1), jnp.float32)] * (n_out - k))
    res = pl.pallas_call(
        functools.partial(_knn_body, k=k, n_src=n_src),
        grid=(n_tgt // tb,),
        in_specs=[
            pl.BlockSpec((tb, 8), lambda i: (i, 0)),
            pl.BlockSpec((8, n_src), lambda i: (0, 0)),
            pl.BlockSpec((8, n_src), lambda i: (0, 0)),
        ],
        out_specs=[pl.BlockSpec((tb, 1), lambda i: (i, 0))] * n_out,
        out_shape=outs,
    )(tpos, spos, sbatch)
    return [r.reshape(n_tgt) for r in res]


def _knn_seg_body(tpos_ref, spos_ref, sbatch_ref, *refs, ncmax, sc):
    out_refs = refs[:6]
    d_s, i_s = refs[6], refs[7]
    j = pl.program_id(1)
    tb_col0 = tpos_ref[:, 3:4]
    sb0 = sbatch_ref[0:1, :]

    @pl.when(j == 0)
    def _init():
        d_s[...] = jnp.full(d_s.shape, jnp.inf, jnp.float32)
        i_s[...] = jnp.zeros(i_s.shape, jnp.int32)

    active = ((jnp.min(sb0) <= jnp.max(tb_col0))
              & (jnp.max(sb0) >= jnp.min(tb_col0)))

    @pl.when(active)
    def _work():
        t = tpos_ref[...]
        tb_col = t[:, 3:4]
        s = spos_ref[...]
        sb = sbatch_ref[0:1, :]
        cross = _cross_term(t, s)
        ss_t = (t[:, 0:1] * t[:, 0:1] + t[:, 1:2] * t[:, 1:2]
                + t[:, 2:3] * t[:, 2:3])
        ss_s = (s[0:1, :] * s[0:1, :] + s[1:2, :] * s[1:2, :]
                + s[2:3, :] * s[2:3, :])
        d2 = jnp.maximum((ss_t + ss_s) - 2.0 * cross, 0.0)
        d2 = jnp.where(tb_col != sb, jnp.float32(jnp.inf), d2)
        gidx = j * sc + lax.broadcasted_iota(jnp.int32, d2.shape, 1)
        dp = jnp.concatenate([d_s[...], d2], axis=1)
        ip = jnp.concatenate([i_s[...], gidx], axis=1)
        ms, sels = [], []
        for _ in range(3):
            m = jnp.min(dp, axis=1, keepdims=True)
            sel = jnp.min(jnp.where(dp == m, ip, jnp.int32(1 << 30)),
                          axis=1, keepdims=True)
            ms.append(m)
            sels.append(sel)
            dp = jnp.where(ip == sel, jnp.float32(jnp.inf), dp)
        nrow = t.shape[0]
        d_s[...] = jnp.concatenate(
            ms + [jnp.full((nrow, 5), jnp.inf, jnp.float32)], axis=1)
        i_s[...] = jnp.concatenate(
            sels + [jnp.zeros((nrow, 5), jnp.int32)], axis=1)

    @pl.when(j == ncmax - 1)
    def _fin():
        dv = d_s[...]
        iv = i_s[...]
        ws = [1.0 / jnp.maximum(dv[:, jj:jj + 1], 1e-16) for jj in range(3)]
        tot = ws[0] + ws[1] + ws[2]
        for jj in range(3):
            out_refs[jj][...] = iv[:, jj:jj + 1]
            out_refs[3 + jj][...] = ws[jj] / tot


def _knn_seg(tpos, spos, sbatch, tb, sc):
    n_tgt = tpos.shape[0]
    n_src = spos.shape[1]
    nb = n_tgt // tb
    ncmax = n_src // sc
    outs = ([jax.ShapeDtypeStruct((n_tgt, 1), jnp.int32)] * 3
            + [jax.ShapeDtypeStruct((n_tgt, 1), jnp.float32)] * 3)
    res = pl.pallas_call(
        functools.partial(_knn_seg_body, ncmax=ncmax, sc=sc),
        grid=(nb, ncmax),
        in_specs=[
            pl.BlockSpec((tb, 8), lambda i, j: (i, 0)),
            pl.BlockSpec((8, sc), lambda i, j: (0, j)),
            pl.BlockSpec((8, sc), lambda i, j: (0, j)),
        ],
        out_specs=[pl.BlockSpec((tb, 1), lambda i, j: (i, 0))] * 6,
        out_shape=outs,
        scratch_shapes=[
            pltpu.VMEM((tb, 8), jnp.float32),
            pltpu.VMEM((tb, 8), jnp.int32),
        ],
    )(tpos, spos, sbatch)
    return [r.reshape(n_tgt) for r in res]


def _tgt_pos(pos, batch):
    n = pos.shape[0]
    b = batch.astype(jnp.float32).reshape(n, 1)
    z = jnp.zeros((n, 4), jnp.float32)
    return jnp.concatenate([pos, b, z], axis=1)


def _src_pos(pos, batch):
    n = pos.shape[0]
    z = jnp.zeros((n, 5), jnp.float32)
    sp = jnp.concatenate([pos, z], axis=1).T
    sb = jnp.broadcast_to(batch.astype(jnp.float32).reshape(1, n), (8, n))
    return sp, sb


def _gather_body(x_hbm, *refs, k, bpw, chunk):
    idx_hbms = refs[:k]
    out_hbms = refs[k:2 * k]
    idx_v, rows_v, sem = refs[2 * k:]
    wid = lax.axis_index("s") * 2 + lax.axis_index("c")
    base = wid * bpw
    for j in range(k):
        pltpu.sync_copy(idx_hbms[j].at[pl.ds(base, bpw)], idx_v)
        for c in range(0, bpw, chunk):
            if bpw == chunk:
                src = x_hbm.at[idx_v]
            else:
                src = x_hbm.at[idx_v.at[pl.ds(c, chunk)]]
            pltpu.async_copy(src, rows_v, sem).wait()
            pltpu.sync_copy(rows_v, out_hbms[j].at[pl.ds(base + c, chunk)])


def _gather_rows(x_src, idxs):
    n_src, ch = x_src.shape
    n_tgt = idxs[0].shape[0]
    k = len(idxs)
    bpw = n_tgt // _NUM_WORKERS
    chunk = min(bpw, 128)
    mesh = plsc.VectorSubcoreMesh(core_axis_name="c", subcore_axis_name="s")
    fn = pl.kernel(
        functools.partial(_gather_body, k=k, bpw=bpw, chunk=chunk),
        out_type=[jax.ShapeDtypeStruct((n_tgt, ch), jnp.float32)] * k,
        mesh=mesh,
        scratch_types=[
            pltpu.VMEM((bpw,), jnp.int32),
            pltpu.VMEM((chunk, ch), jnp.float32),
            pltpu.SemaphoreType.DMA,
        ],
    )
    return fn(x_src, *idxs)


def _mlp_body(w1a_ref, w1b_ref, b1_ref, w2_ref, b2_ref, skip_ref, *refs, k):
    bufs = refs[:k]
    ws = refs[k:2 * k - 1] if k == 1 else refs[k:2 * k]
    out_ref = refs[-1]
    acc = bufs[0][...] if k == 1 else ws[0][...] * bufs[0][...]
    for j in range(1, k):
        acc = acc + ws[j][...] * bufs[j][...]
    h = (jnp.dot(acc, w1a_ref[...], preferred_element_type=jnp.float32)
         + jnp.dot(skip_ref[...], w1b_ref[...],
                   preferred_element_type=jnp.float32)
         + b1_ref[...])
    h = jnp.maximum(h, 0.0)
    out_ref[...] = (jnp.dot(h, w2_ref[...], preferred_element_type=jnp.float32)
                    + b2_ref[...])


def _mlp(bufs, ws, skip, W1, b1, W2, b2, rb):
    n, cin = bufs[0].shape
    cskip = skip.shape[1]
    f2 = W1.shape[1]
    f3 = W2.shape[1]
    k = len(bufs)
    W1a = W1[:cin]
    W1b = W1[cin:]
    if cskip % 8 != 0:
        pad = 8 - cskip % 8
        skip = jnp.concatenate([skip, jnp.zeros((n, pad), jnp.float32)], 1)
        W1b = jnp.concatenate([W1b, jnp.zeros((pad, f2), jnp.float32)], 0)
        cskip += pad
    grid = n // rb
    fixed = lambda i: (0, 0)
    row = lambda i: (i, 0)
    in_specs = [
        pl.BlockSpec((cin, f2), fixed),
        pl.BlockSpec((cskip, f2), fixed),
        pl.BlockSpec((1, f2), fixed),
        pl.BlockSpec((f2, f3), fixed),
        pl.BlockSpec((1, f3), fixed),
        pl.BlockSpec((rb, cskip), row),
    ]
    in_specs += [pl.BlockSpec((rb, cin), row)] * k
    in_specs += [pl.BlockSpec((rb, 1), row)] * len(ws)
    args = [W1a, W1b, b1.reshape(1, f2), W2, b2.reshape(1, f3), skip]
    args += list(bufs)
    args += [w.reshape(n, 1) for w in ws]
    return pl.pallas_call(
        functools.partial(_mlp_body, k=k),
        grid=(grid,),
        in_specs=in_specs,
        out_specs=pl.BlockSpec((rb, f3), row),
        out_shape=jax.ShapeDtypeStruct((n, f3), jnp.float32),
    )(*args)


def kernel(sa0_x, sa0_pos, sa0_batch, sa1_x, sa1_pos, sa1_batch, sa2_x,
           sa2_pos, sa2_batch, sa3_x, sa3_pos, sa3_batch, sa4_x, sa4_pos,
           sa4_batch, fp4_W1, fp4_b1, fp4_W2, fp4_b2, fp3_W1, fp3_b1,
           fp3_W2, fp3_b2, fp2_W1, fp2_b1, fp2_W2, fp2_b2, fp1_W1, fp1_b1,
           fp1_W2, fp1_b2):
    t0 = _tgt_pos(sa0_pos, sa0_batch)
    t1 = _tgt_pos(sa1_pos, sa1_batch)
    t2 = _tgt_pos(sa2_pos, sa2_batch)
    t3 = _tgt_pos(sa3_pos, sa3_batch)
    s1p, s1b = _src_pos(sa1_pos, sa1_batch)
    s2p, s2b = _src_pos(sa2_pos, sa2_batch)
    s3p, s3b = _src_pos(sa3_pos, sa3_batch)
    s4p, s4b = _src_pos(sa4_pos, sa4_batch)

    knn4 = _knn(t3, s4p, s4b, k=1, tb=256)
    knn3 = _knn(t2, s3p, s3b, k=3, tb=1024)
    knn2 = _knn(t1, s2p, s2b, k=3, tb=512)
    knn1 = _knn_seg(t0, s1p, s1b, tb=1024, sc=1024)

    (buf,) = _gather_rows(sa4_x, knn4[:1])
    x = _mlp([buf], [], sa3_x, fp4_W1, fp4_b1, fp4_W2, fp4_b2, rb=256)

    bufs = _gather_rows(x, knn3[:3])
    x = _mlp(bufs, knn3[3:], sa2_x, fp3_W1, fp3_b1, fp3_W2, fp3_b2, rb=1024)

    bufs = _gather_rows(x, knn2[:3])
    x = _mlp(bufs, knn2[3:], sa1_x, fp2_W1, fp2_b1, fp2_W2, fp2_b2, rb=2048)

    bufs = _gather_rows(x, knn1[:3])
    x = _mlp(bufs, knn1[3:], sa0_x, fp1_W1, fp1_b1, fp1_W2, fp1_b2, rb=2048)

    return (x, sa0_pos, sa0_batch)


# speedup vs baseline: 1.0364x; 1.0364x over previous
import functools

import jax
import jax.numpy as jnp
from jax import lax
from jax.experimental import pallas as pl
from jax.experimental.pallas import tpu as pltpu
from jax.experimental.pallas import tpu_sc as plsc

_NUM_WORKERS = 32


def _knn_body(tpos_ref, spos_ref, sbatch_ref, *out_refs, k, n_src):
    t = tpos_ref[...]
    s = spos_ref[...]
    sb = sbatch_ref[0:1, :]
    cross = jnp.dot(t, s, preferred_element_type=jnp.float32)
    ss_t = t[:, 0:1] * t[:, 0:1] + t[:, 1:2] * t[:, 1:2] + t[:, 2:3] * t[:, 2:3]
    ss_s = s[0:1, :] * s[0:1, :] + s[1:2, :] * s[1:2, :] + s[2:3, :] * s[2:3, :]
    d2 = (ss_t + ss_s) - 2.0 * cross
    d2 = jnp.maximum(d2, 0.0)
    d2 = jnp.where(t[:, 3:4] != sb, jnp.float32(jnp.inf), d2)
    lanes = lax.broadcasted_iota(jnp.int32, d2.shape, 1)
    dists = []
    for j in range(k):
        m = jnp.min(d2, axis=1, keepdims=True)
        eq = d2 == m
        idx = jnp.min(jnp.where(eq, lanes, n_src), axis=1, keepdims=True)
        out_refs[j][...] = idx
        dists.append(m)
        if j + 1 < k:
            d2 = jnp.where(lanes == idx, jnp.float32(jnp.inf), d2)
    if k > 1:
        ws = [1.0 / jnp.maximum(d, 1e-16) for d in dists]
        tot = ws[0]
        for w in ws[1:]:
            tot = tot + w
        for j in range(k):
            out_refs[k + j][...] = ws[j] / tot


def _knn(tpos, spos, sbatch, k, tb):
    n_tgt = tpos.shape[0]
    n_src = spos.shape[1]
    n_out = 2 * k if k > 1 else 1
    outs = ([jax.ShapeDtypeStruct((n_tgt, 1), jnp.int32)] * k
            + [jax.ShapeDtypeStruct((n_tgt, 1), jnp.float32)] * (n_out - k))
    res = pl.pallas_call(
        functools.partial(_knn_body, k=k, n_src=n_src),
        grid=(n_tgt // tb,),
        in_specs=[
            pl.BlockSpec((tb, 8), lambda i: (i, 0)),
            pl.BlockSpec((8, n_src), lambda i: (0, 0)),
            pl.BlockSpec((8, n_src), lambda i: (0, 0)),
        ],
        out_specs=[pl.BlockSpec((tb, 1), lambda i: (i, 0))] * n_out,
        out_shape=outs,
    )(tpos, spos, sbatch)
    return [r.reshape(n_tgt) for r in res]


def _knn_seg_body(tpos_ref, spos_ref, sbatch_ref, *refs, ncmax, sc):
    out_refs = refs[:6]
    d_s, i_s = refs[6], refs[7]
    j = pl.program_id(1)
    tb_col0 = tpos_ref[:, 3:4]
    sb0 = sbatch_ref[0:1, :]

    @pl.when(j == 0)
    def _init():
        d_s[...] = jnp.full(d_s.shape, jnp.inf, jnp.float32)
        i_s[...] = jnp.zeros(i_s.shape, jnp.int32)

    active = ((jnp.min(sb0) <= jnp.max(tb_col0))
              & (jnp.max(sb0) >= jnp.min(tb_col0)))

    @pl.when(active)
    def _work():
        t = tpos_ref[...]
        tb_col = t[:, 3:4]
        s = spos_ref[...]
        sb = sbatch_ref[0:1, :]
        cross = jnp.dot(t, s, preferred_element_type=jnp.float32)
        ss_t = (t[:, 0:1] * t[:, 0:1] + t[:, 1:2] * t[:, 1:2]
                + t[:, 2:3] * t[:, 2:3])
        ss_s = (s[0:1, :] * s[0:1, :] + s[1:2, :] * s[1:2, :]
                + s[2:3, :] * s[2:3, :])
        d2 = jnp.maximum((ss_t + ss_s) - 2.0 * cross, 0.0)
        d2 = jnp.where(tb_col != sb, jnp.float32(jnp.inf), d2)
        gidx = j * sc + lax.broadcasted_iota(jnp.int32, d2.shape, 1)
        dp = jnp.concatenate([d_s[...], d2], axis=1)
        ip = jnp.concatenate([i_s[...], gidx], axis=1)
        ms, sels = [], []
        for _ in range(3):
            m = jnp.min(dp, axis=1, keepdims=True)
            sel = jnp.min(jnp.where(dp == m, ip, jnp.int32(1 << 30)),
                          axis=1, keepdims=True)
            ms.append(m)
            sels.append(sel)
            dp = jnp.where(ip == sel, jnp.float32(jnp.inf), dp)
        nrow = t.shape[0]
        d_s[...] = jnp.concatenate(
            ms + [jnp.full((nrow, 5), jnp.inf, jnp.float32)], axis=1)
        i_s[...] = jnp.concatenate(
            sels + [jnp.zeros((nrow, 5), jnp.int32)], axis=1)

    @pl.when(j == ncmax - 1)
    def _fin():
        dv = d_s[...]
        iv = i_s[...]
        ws = [1.0 / jnp.maximum(dv[:, jj:jj + 1], 1e-16) for jj in range(3)]
        tot = ws[0] + ws[1] + ws[2]
        for jj in range(3):
            out_refs[jj][...] = iv[:, jj:jj + 1]
            out_refs[3 + jj][...] = ws[jj] / tot


def _knn_seg(tpos, spos, sbatch, tb, sc):
    n_tgt = tpos.shape[0]
    n_src = spos.shape[1]
    nb = n_tgt // tb
    ncmax = n_src // sc
    outs = ([jax.ShapeDtypeStruct((n_tgt, 1), jnp.int32)] * 3
            + [jax.ShapeDtypeStruct((n_tgt, 1), jnp.float32)] * 3)
    res = pl.pallas_call(
        functools.partial(_knn_seg_body, ncmax=ncmax, sc=sc),
        grid=(nb, ncmax),
        in_specs=[
            pl.BlockSpec((tb, 8), lambda i, j: (i, 0)),
            pl.BlockSpec((8, sc), lambda i, j: (0, j)),
            pl.BlockSpec((8, sc), lambda i, j: (0, j)),
        ],
        out_specs=[pl.BlockSpec((tb, 1), lambda i, j: (i, 0))] * 6,
        out_shape=outs,
        scratch_shapes=[
            pltpu.VMEM((tb, 8), jnp.float32),
            pltpu.VMEM((tb, 8), jnp.int32),
        ],
    )(tpos, spos, sbatch)
    return [r.reshape(n_tgt) for r in res]


def _tgt_pos(pos, batch):
    n = pos.shape[0]
    b = batch.astype(jnp.float32).reshape(n, 1)
    z = jnp.zeros((n, 4), jnp.float32)
    return jnp.concatenate([pos, b, z], axis=1)


def _src_pos(pos, batch):
    n = pos.shape[0]
    z = jnp.zeros((n, 5), jnp.float32)
    sp = jnp.concatenate([pos, z], axis=1).T
    sb = jnp.broadcast_to(batch.astype(jnp.float32).reshape(1, n), (8, n))
    return sp, sb


def _gather_body(x_hbm, *refs, k, bpw, chunk):
    idx_hbms = refs[:k]
    out_hbms = refs[k:2 * k]
    idx_v, rows_v, sem = refs[2 * k:]
    wid = lax.axis_index("s") * 2 + lax.axis_index("c")
    base = wid * bpw
    for j in range(k):
        pltpu.sync_copy(idx_hbms[j].at[pl.ds(base, bpw)], idx_v)
        for c in range(0, bpw, chunk):
            if bpw == chunk:
                src = x_hbm.at[idx_v]
            else:
                src = x_hbm.at[idx_v.at[pl.ds(c, chunk)]]
            pltpu.async_copy(src, rows_v, sem).wait()
            pltpu.sync_copy(rows_v, out_hbms[j].at[pl.ds(base + c, chunk)])


def _gather_rows(x_src, idxs):
    n_src, ch = x_src.shape
    n_tgt = idxs[0].shape[0]
    k = len(idxs)
    bpw = n_tgt // _NUM_WORKERS
    chunk = min(bpw, 128)
    mesh = plsc.VectorSubcoreMesh(core_axis_name="c", subcore_axis_name="s")
    fn = pl.kernel(
        functools.partial(_gather_body, k=k, bpw=bpw, chunk=chunk),
        out_type=[jax.ShapeDtypeStruct((n_tgt, ch), jnp.float32)] * k,
        mesh=mesh,
        scratch_types=[
            pltpu.VMEM((bpw,), jnp.int32),
            pltpu.VMEM((chunk, ch), jnp.float32),
            pltpu.SemaphoreType.DMA,
        ],
    )
    return fn(x_src, *idxs)


def _mlp_body(w1a_ref, w1b_ref, b1_ref, w2_ref, b2_ref, skip_ref, *refs, k):
    bufs = refs[:k]
    ws = refs[k:2 * k - 1] if k == 1 else refs[k:2 * k]
    out_ref = refs[-1]
    acc = bufs[0][...] if k == 1 else ws[0][...] * bufs[0][...]
    for j in range(1, k):
        acc = acc + ws[j][...] * bufs[j][...]
    h = (jnp.dot(acc, w1a_ref[...], preferred_element_type=jnp.float32)
         + jnp.dot(skip_ref[...], w1b_ref[...],
                   preferred_element_type=jnp.float32)
         + b1_ref[...])
    h = jnp.maximum(h, 0.0)
    out_ref[...] = (jnp.dot(h, w2_ref[...], preferred_element_type=jnp.float32)
                    + b2_ref[...])


def _mlp(bufs, ws, skip, W1, b1, W2, b2, rb):
    n, cin = bufs[0].shape
    cskip = skip.shape[1]
    f2 = W1.shape[1]
    f3 = W2.shape[1]
    k = len(bufs)
    W1a = W1[:cin]
    W1b = W1[cin:]
    if cskip % 8 != 0:
        pad = 8 - cskip % 8
        skip = jnp.concatenate([skip, jnp.zeros((n, pad), jnp.float32)], 1)
        W1b = jnp.concatenate([W1b, jnp.zeros((pad, f2), jnp.float32)], 0)
        cskip += pad
    grid = n // rb
    fixed = lambda i: (0, 0)
    row = lambda i: (i, 0)
    in_specs = [
        pl.BlockSpec((cin, f2), fixed),
        pl.BlockSpec((cskip, f2), fixed),
        pl.BlockSpec((1, f2), fixed),
        pl.BlockSpec((f2, f3), fixed),
        pl.BlockSpec((1, f3), fixed),
        pl.BlockSpec((rb, cskip), row),
    ]
    in_specs += [pl.BlockSpec((rb, cin), row)] * k
    in_specs += [pl.BlockSpec((rb, 1), row)] * len(ws)
    args = [W1a, W1b, b1.reshape(1, f2), W2, b2.reshape(1, f3), skip]
    args += list(bufs)
    args += [w.reshape(n, 1) for w in ws]
    return pl.pallas_call(
        functools.partial(_mlp_body, k=k),
        grid=(grid,),
        in_specs=in_specs,
        out_specs=pl.BlockSpec((rb, f3), row),
        out_shape=jax.ShapeDtypeStruct((n, f3), jnp.float32),
    )(*args)


def kernel(sa0_x, sa0_pos, sa0_batch, sa1_x, sa1_pos, sa1_batch, sa2_x,
           sa2_pos, sa2_batch, sa3_x, sa3_pos, sa3_batch, sa4_x, sa4_pos,
           sa4_batch, fp4_W1, fp4_b1, fp4_W2, fp4_b2, fp3_W1, fp3_b1,
           fp3_W2, fp3_b2, fp2_W1, fp2_b1, fp2_W2, fp2_b2, fp1_W1, fp1_b1,
           fp1_W2, fp1_b2):
    t0 = _tgt_pos(sa0_pos, sa0_batch)
    t1 = _tgt_pos(sa1_pos, sa1_batch)
    t2 = _tgt_pos(sa2_pos, sa2_batch)
    t3 = _tgt_pos(sa3_pos, sa3_batch)
    s1p, s1b = _src_pos(sa1_pos, sa1_batch)
    s2p, s2b = _src_pos(sa2_pos, sa2_batch)
    s3p, s3b = _src_pos(sa3_pos, sa3_batch)
    s4p, s4b = _src_pos(sa4_pos, sa4_batch)

    knn4 = _knn(t3, s4p, s4b, k=1, tb=256)
    knn3 = _knn(t2, s3p, s3b, k=3, tb=1024)
    knn2 = _knn(t1, s2p, s2b, k=3, tb=512)
    knn1 = _knn_seg(t0, s1p, s1b, tb=1024, sc=1024)

    (buf,) = _gather_rows(sa4_x, knn4[:1])
    x = _mlp([buf], [], sa3_x, fp4_W1, fp4_b1, fp4_W2, fp4_b2, rb=256)

    bufs = _gather_rows(x, knn3[:3])
    x = _mlp(bufs, knn3[3:], sa2_x, fp3_W1, fp3_b1, fp3_W2, fp3_b2, rb=1024)

    bufs = _gather_rows(x, knn2[:3])
    x = _mlp(bufs, knn2[3:], sa1_x, fp2_W1, fp2_b1, fp2_W2, fp2_b2, rb=2048)

    bufs = _gather_rows(x, knn1[:3])
    x = _mlp(bufs, knn1[3:], sa0_x, fp1_W1, fp1_b1, fp1_W2, fp1_b2, rb=2048)

    return (x, sa0_pos, sa0_batch)
